# baseline (speedup 1.0000x reference)
import functools

import jax
import jax.numpy as jnp
from jax import lax
from jax.experimental import pallas as pl
from jax.experimental.pallas import tpu as pltpu
from jax.experimental.pallas import tpu_sc as plsc

_B = 128
_S = 197
_LP = 20
_D = 768
_NPOOL = 30
_PROW = _LP * _D
_XROW = _S * _D
_OROW = _PROW + _XROW
_NC = 2
_NS = 16
_NW = _NC * _NS
_BPW = _B // _NW
_NCHUNK = 6
_XCHUNK = _XROW // _NCHUNK


def _sc_concat(x2, idx2, p2):
    mesh = plsc.VectorSubcoreMesh(core_axis_name="c", subcore_axis_name="s")

    @functools.partial(
        pl.kernel,
        mesh=mesh,
        out_type=jax.ShapeDtypeStruct((_B, _OROW), jnp.float32),
        scratch_types=[
            pltpu.VMEM((_BPW,), jnp.int32),
            pltpu.VMEM((_BPW, _PROW), jnp.float32),
            pltpu.VMEM((2, _XCHUNK), jnp.float32),
            pltpu.SemaphoreType.DMA,
        ],
    )
    def body(x_hbm, idx_hbm, p_hbm, out_hbm, idx_v, pbuf, xbuf, sem):
        wid = lax.axis_index("s") * _NC + lax.axis_index("c")
        base = wid * _BPW
        pltpu.sync_copy(idx_hbm.at[wid], idx_v)
        pltpu.async_copy(p_hbm.at[idx_v], pbuf, sem).wait()
        for i in range(_BPW):
            b = base + i
            pltpu.sync_copy(pbuf.at[i], out_hbm.at[b, pl.ds(0, _PROW)])
            for c in range(_NCHUNK):
                buf = xbuf.at[c % 2]
                pltpu.sync_copy(x_hbm.at[b, pl.ds(c * _XCHUNK, _XCHUNK)], buf)
                pltpu.sync_copy(
                    buf, out_hbm.at[b, pl.ds(_PROW + c * _XCHUNK, _XCHUNK)]
                )

    return body(x2, idx2, p2)


def kernel(x, prompt_idx, prompts):
    x2 = x.reshape(_B, _XROW)
    idx2 = prompt_idx.astype(jnp.int32).reshape(_NW, _BPW)
    p2 = prompts.reshape(_NPOOL, _PROW)
    out = _sc_concat(x2, idx2, p2)
    return out.reshape(_B, _LP + _S, _D)
